# baseline (speedup 1.0000x reference)
import functools
import math

import jax
import jax.numpy as jnp
from jax.experimental import pallas as pl

N = 100000
E = 1600000
R_MAX = 4.0
NB = 8
NT = 4
DS = 32
AVG = 20.0

EPAD = 1638400
BT = 1024


def _silu(x):
    return x / (1.0 + jnp.exp(-x))


def _edge_mlp_body(r2_ref, pid_ref, w0b_ref, pt_ref, b0_ref, w1_ref, b1_ref, out_ref):
    i = pl.program_id(0)
    r2 = r2_ref[...]
    r = jnp.sqrt(r2 + 1e-12)
    x = r * (1.0 / R_MAX)
    n = jax.lax.broadcasted_iota(jnp.int32, (BT, NB), 1).astype(jnp.float32) + 1.0
    arg = n * (jnp.pi * x)
    bes = math.sqrt(2.0 / R_MAX) * jnp.sin(arg) / (r + 1e-9)
    p = 6.0
    x2 = x * x
    x3 = x2 * x
    x6 = x3 * x3
    cut = 1.0 - (p + 1.0) * (p + 2.0) / 2.0 * x6 + p * (p + 2.0) * x6 * x - p * (p + 1.0) / 2.0 * x6 * x2
    cut = jnp.where(x < 1.0, cut, 0.0)
    bes = bes * cut
    pid = pid_ref[...]
    onehot = (jax.lax.broadcasted_iota(jnp.int32, (BT, NT * NT), 1) == pid).astype(jnp.float32)
    h = jnp.dot(bes, w0b_ref[...], preferred_element_type=jnp.float32)
    h = h + jnp.dot(onehot, pt_ref[...], preferred_element_type=jnp.float32)
    h = _silu(h + b0_ref[...])
    ef = _silu(jnp.dot(h, w1_ref[...], preferred_element_type=jnp.float32) + b1_ref[...])
    rows = i * BT + jax.lax.broadcasted_iota(jnp.int32, (BT, 1), 0)
    out_ref[...] = jnp.where(rows < E, ef, 0.0)


def _layer_body(ef_ref, env_ref, a1_ref, a2_ref, lb_ref, lc_ref, out_ref):
    i = pl.program_id(0)
    ef = ef_ref[...]
    z = _silu(jnp.dot(ef, a1_ref[...], preferred_element_type=jnp.float32)
              + jnp.dot(env_ref[...], a2_ref[...], preferred_element_type=jnp.float32))
    z = _silu(jnp.dot(z, lb_ref[...], preferred_element_type=jnp.float32))
    z = _silu(jnp.dot(z, lc_ref[...], preferred_element_type=jnp.float32))
    out = ef + z
    rows = i * BT + jax.lax.broadcasted_iota(jnp.int32, (BT, 1), 0)
    out_ref[...] = jnp.where(rows < E, out, 0.0)


def _readout_body(ef_ref, wr0_ref, wr1_ref, out_ref):
    i = pl.program_id(0)
    s = _silu(jnp.dot(ef_ref[...], wr0_ref[...], preferred_element_type=jnp.float32))
    e = jnp.dot(s, wr1_ref[...], preferred_element_type=jnp.float32)
    rows = i * BT + jax.lax.broadcasted_iota(jnp.int32, (BT, 1), 0)
    out_ref[...] = jnp.where(rows < E, e, 0.0)


def _full(shape):
    return pl.BlockSpec(shape, lambda i: (0, 0))


def _edge_mlp(r2, pid, w0b, pt, b0, w1, b1):
    g = EPAD // BT
    return pl.pallas_call(
        _edge_mlp_body,
        grid=(g,),
        in_specs=[
            pl.BlockSpec((BT, 1), lambda i: (i, 0)),
            pl.BlockSpec((BT, 1), lambda i: (i, 0)),
            _full((NB, DS)),
            _full((NT * NT, DS)),
            _full((1, DS)),
            _full((DS, DS)),
            _full((1, DS)),
        ],
        out_specs=pl.BlockSpec((BT, DS), lambda i: (i, 0)),
        out_shape=jax.ShapeDtypeStruct((EPAD, DS), jnp.float32),
    )(r2, pid, w0b, pt, b0, w1, b1)


def _layer(ef, env, a1, a2, lb, lc):
    g = EPAD // BT
    return pl.pallas_call(
        _layer_body,
        grid=(g,),
        in_specs=[
            pl.BlockSpec((BT, DS), lambda i: (i, 0)),
            pl.BlockSpec((BT, DS), lambda i: (i, 0)),
            _full((DS, DS)),
            _full((DS, DS)),
            _full((DS, DS)),
            _full((DS, DS)),
        ],
        out_specs=pl.BlockSpec((BT, DS), lambda i: (i, 0)),
        out_shape=jax.ShapeDtypeStruct((EPAD, DS), jnp.float32),
    )(ef, env, a1, a2, lb, lc)


def _readout(ef, wr0, wr1):
    g = EPAD // BT
    return pl.pallas_call(
        _readout_body,
        grid=(g,),
        in_specs=[
            pl.BlockSpec((BT, DS), lambda i: (i, 0)),
            _full((DS, 8)),
            _full((8, 1)),
        ],
        out_specs=pl.BlockSpec((BT, 1), lambda i: (i, 0)),
        out_shape=jax.ShapeDtypeStruct((EPAD, 1), jnp.float32),
    )(ef, wr0, wr1)


def kernel(positions, atom_types, edge_index, W_pair, W0, b0, W1, b1,
           L0a, L0b, L0c, L1a, L1b, L1c, Wr0, Wr1):
    src = edge_index[0]
    dst = edge_index[1]
    pad = EPAD - E
    srcp = jnp.concatenate([src, jnp.zeros((pad,), src.dtype)])
    dstp = jnp.concatenate([dst, jnp.zeros((pad,), dst.dtype)])

    vec = positions[dstp] - positions[srcp]
    r2 = jnp.sum(vec * vec, axis=-1, keepdims=True)
    pid = (atom_types[srcp] * NT + atom_types[dstp]).astype(jnp.int32)[:, None]

    w0b = W0[:NB]
    pt = W_pair @ W0[NB:]
    b0r = b0[None, :]
    b1r = b1[None, :]

    ef = _edge_mlp(r2, pid, w0b, pt, b0r, W1, b1r)

    for (Wa, Wb, Wc) in ((L0a, L0b, L0c), (L1a, L1b, L1c)):
        a1 = Wa[:DS]
        a2 = Wa[DS:] * (1.0 / AVG)
        node_env = jax.ops.segment_sum(ef, dstp, num_segments=N)
        env = node_env[srcp]
        ef = _layer(ef, env, a1, a2, Wb, Wc)

    e = _readout(ef, Wr0, Wr1)[:, 0]
    atom_energy = jax.ops.segment_sum(e, dstp, num_segments=N) * (1.0 / math.sqrt(AVG))
    return atom_energy


# speedup vs baseline: 4.1362x; 4.1362x over previous
import functools
import math

import jax
import jax.numpy as jnp
from jax import lax
from jax.experimental import pallas as pl
from jax.experimental.pallas import tpu as pltpu
from jax.experimental.pallas import tpu_sc as plsc

N = 100000
E = 1600000
R_MAX = 4.0
NB = 8
NT = 4
DS = 32
AVG = 20.0

EPAD = 1638400
ROWS = EPAD // 128
BT = 1024

_NC = 2
_NS = 16
_NW = _NC * _NS

HALF = N // 2
SPAD_ROWS = 50048
NE_PAD = 100352


def _silu(x):
    return x / (1.0 + jnp.exp(-x))


def _sc_gather(table, idx2, d, chr_):
    rows = idx2.shape[0]
    rows_per_w = rows // _NW
    n_macro = rows_per_w // chr_
    mesh = plsc.VectorSubcoreMesh(core_axis_name="c", subcore_axis_name="s")

    def body(table_ref, idx_ref, out_ref, idx_v, rows_v, sem):
        wid = lax.axis_index("s") * _NC + lax.axis_index("c")
        base = wid * rows_per_w

        def macro(m, carry):
            rb = base + m * chr_
            pltpu.sync_copy(idx_ref.at[pl.ds(rb, chr_)], idx_v)
            handles = []
            for j in range(chr_):
                handles.append(pltpu.async_copy(table_ref.at[idx_v.at[j]], rows_v.at[j], sem))
            for h in handles:
                h.wait()
            pltpu.sync_copy(rows_v, out_ref.at[pl.ds(rb, chr_)])
            return carry

        lax.fori_loop(0, n_macro, macro, 0)

    f = pl.kernel(
        body,
        out_type=jax.ShapeDtypeStruct((rows, 128, d), jnp.float32),
        mesh=mesh,
        compiler_params=pltpu.CompilerParams(use_tc_tiling_on_sc=False),
        scratch_types=[
            pltpu.VMEM((chr_, 128), jnp.int32),
            pltpu.VMEM((chr_, 128, d), jnp.float32),
            pltpu.SemaphoreType.DMA,
        ],
    )
    return f(table, idx2)


def _sc_scatter_env(ef3, dst2, zrows):
    chr_ = 4
    rows_per_tile = ROWS // _NS
    n_macro = rows_per_tile // chr_
    z_per_tile = SPAD_ROWS // _NS
    o_per_tile = HALF // _NS
    mesh = plsc.VectorSubcoreMesh(core_axis_name="c", subcore_axis_name="s")

    def body(ef_ref, dst_ref, z_ref, out_ref, idx_v, ef_v, acc, sem):
        c = lax.axis_index("c")
        s = lax.axis_index("s")
        pltpu.sync_copy(z_ref.at[pl.ds(0, z_per_tile)], acc.at[pl.ds(s * z_per_tile, z_per_tile)])
        plsc.subcore_barrier()
        lo = c * HALF

        def macro(m, carry):
            rb = s * rows_per_tile + m * chr_
            pltpu.sync_copy(dst_ref.at[pl.ds(rb, chr_)], idx_v)
            pltpu.sync_copy(ef_ref.at[pl.ds(rb, chr_)], ef_v)
            for j in range(chr_):
                for k in range(8):
                    lid = idx_v[j, pl.ds(k * 16, 16)] - lo
                    ok = (lid >= 0) & (lid < HALF)
                    idx_v[j, pl.ds(k * 16, 16)] = jnp.where(ok, lid, HALF)
            handles = []
            for j in range(chr_):
                handles.append(pltpu.async_copy(ef_v.at[j], acc.at[idx_v.at[j]], sem, add=True))
            for h in handles:
                h.wait()
            return carry

        lax.fori_loop(0, n_macro, macro, 0)
        plsc.subcore_barrier()
        pltpu.sync_copy(acc.at[pl.ds(s * o_per_tile, o_per_tile)],
                        out_ref.at[pl.ds(c * HALF + s * o_per_tile, o_per_tile)])

    f = pl.kernel(
        body,
        out_type=jax.ShapeDtypeStruct((N, DS), jnp.float32),
        mesh=mesh,
        compiler_params=pltpu.CompilerParams(use_tc_tiling_on_sc=False),
        scratch_types=[
            pltpu.VMEM((chr_, 128), jnp.int32),
            pltpu.VMEM((chr_, 128, DS), jnp.float32),
            pltpu.VMEM_SHARED((SPAD_ROWS, DS), jnp.float32),
            pltpu.SemaphoreType.DMA,
        ],
    )
    return f(ef3, dst2, zrows)


def _sc_scatter_energy(e2, dst2, z1):
    chr_ = 16
    rows_per_tile = ROWS // _NW
    n_macro = rows_per_tile // chr_
    z_per_tile = NE_PAD // _NS
    mesh = plsc.VectorSubcoreMesh(core_axis_name="c", subcore_axis_name="s")

    def body(e_ref, dst_ref, z_ref, out_ref, idx_v, e_v, acc, sem):
        c = lax.axis_index("c")
        s = lax.axis_index("s")
        pltpu.sync_copy(z_ref, acc.at[pl.ds(s * z_per_tile, z_per_tile)])
        plsc.subcore_barrier()
        base = (c * _NS + s) * rows_per_tile

        def macro(m, carry):
            rb = base + m * chr_
            pltpu.sync_copy(dst_ref.at[pl.ds(rb, chr_)], idx_v)
            pltpu.sync_copy(e_ref.at[pl.ds(rb, chr_)], e_v)
            handles = []
            for j in range(chr_):
                handles.append(pltpu.async_copy(e_v.at[j], acc.at[idx_v.at[j]], sem, add=True))
            for h in handles:
                h.wait()
            return carry

        lax.fori_loop(0, n_macro, macro, 0)
        plsc.subcore_barrier()
        pltpu.sync_copy(acc.at[pl.ds(s * z_per_tile, z_per_tile)],
                        out_ref.at[c, pl.ds(s * z_per_tile, z_per_tile)])

    f = pl.kernel(
        body,
        out_type=jax.ShapeDtypeStruct((2, NE_PAD), jnp.float32),
        mesh=mesh,
        compiler_params=pltpu.CompilerParams(use_tc_tiling_on_sc=False),
        scratch_types=[
            pltpu.VMEM((chr_, 128), jnp.int32),
            pltpu.VMEM((chr_, 128), jnp.float32),
            pltpu.VMEM_SHARED((NE_PAD,), jnp.float32),
            pltpu.SemaphoreType.DMA,
        ],
    )
    return f(e2, dst2, z1)


def _edge_mlp_body(rs_ref, rd_ref, w0b_ref, pt_ref, b0_ref, w1_ref, b1_ref, out_ref):
    i = pl.program_id(0)
    rs = rs_ref[...]
    rd = rd_ref[...]
    d = rd - rs
    lane = jax.lax.broadcasted_iota(jnp.int32, (BT, 8), 1)
    d2 = jnp.where(lane < 3, d * d, 0.0)
    r2 = jnp.sum(d2, axis=1, keepdims=True)
    ts = jnp.sum(jnp.where(lane == 3, rs, 0.0), axis=1, keepdims=True)
    td = jnp.sum(jnp.where(lane == 3, rd, 0.0), axis=1, keepdims=True)
    pidf = ts * float(NT) + td
    r = jnp.sqrt(r2 + 1e-12)
    x = r * (1.0 / R_MAX)
    n = jax.lax.broadcasted_iota(jnp.int32, (BT, NB), 1).astype(jnp.float32) + 1.0
    arg = n * (jnp.pi * x)
    bes = math.sqrt(2.0 / R_MAX) * jnp.sin(arg) / (r + 1e-9)
    p = 6.0
    x2 = x * x
    x6 = x2 * x2 * x2
    cut = 1.0 - (p + 1.0) * (p + 2.0) / 2.0 * x6 + p * (p + 2.0) * x6 * x - p * (p + 1.0) / 2.0 * x6 * x2
    cut = jnp.where(x < 1.0, cut, 0.0)
    bes = bes * cut
    onehot = (jax.lax.broadcasted_iota(jnp.int32, (BT, NT * NT), 1).astype(jnp.float32) == pidf
              ).astype(jnp.float32)
    h = jnp.dot(bes, w0b_ref[...], preferred_element_type=jnp.float32)
    h = h + jnp.dot(onehot, pt_ref[...], preferred_element_type=jnp.float32)
    h = _silu(h + b0_ref[...])
    ef = _silu(jnp.dot(h, w1_ref[...], preferred_element_type=jnp.float32) + b1_ref[...])
    rows = i * BT + jax.lax.broadcasted_iota(jnp.int32, (BT, 1), 0)
    out_ref[...] = jnp.where(rows < E, ef, 0.0)


def _layer_body(ef_ref, env_ref, a1_ref, a2_ref, lb_ref, lc_ref, out_ref):
    i = pl.program_id(0)
    ef = ef_ref[...]
    z = _silu(jnp.dot(ef, a1_ref[...], preferred_element_type=jnp.float32)
              + jnp.dot(env_ref[...], a2_ref[...], preferred_element_type=jnp.float32))
    z = _silu(jnp.dot(z, lb_ref[...], preferred_element_type=jnp.float32))
    z = _silu(jnp.dot(z, lc_ref[...], preferred_element_type=jnp.float32))
    out = ef + z
    rows = i * BT + jax.lax.broadcasted_iota(jnp.int32, (BT, 1), 0)
    out_ref[...] = jnp.where(rows < E, out, 0.0)


def _readout_body(ef_ref, wr0_ref, wr1_ref, out_ref):
    i = pl.program_id(0)
    s = _silu(jnp.dot(ef_ref[...], wr0_ref[...], preferred_element_type=jnp.float32))
    e = jnp.dot(s, wr1_ref[...], preferred_element_type=jnp.float32)
    rows = i * BT + jax.lax.broadcasted_iota(jnp.int32, (BT, 1), 0)
    out_ref[...] = jnp.where(rows < E, e, 0.0)


def _combine_body(p_ref, out_ref):
    out_ref[...] = (p_ref[0] + p_ref[1]) * (1.0 / math.sqrt(AVG))


def _full(shape):
    return pl.BlockSpec(shape, lambda i: (0, 0))


def _edge_mlp(rows_s, rows_d, w0b, pt, b0, w1, b1):
    g = EPAD // BT
    return pl.pallas_call(
        _edge_mlp_body,
        grid=(g,),
        in_specs=[
            pl.BlockSpec((BT, 8), lambda i: (i, 0)),
            pl.BlockSpec((BT, 8), lambda i: (i, 0)),
            _full((NB, DS)),
            _full((NT * NT, DS)),
            _full((1, DS)),
            _full((DS, DS)),
            _full((1, DS)),
        ],
        out_specs=pl.BlockSpec((BT, DS), lambda i: (i, 0)),
        out_shape=jax.ShapeDtypeStruct((EPAD, DS), jnp.float32),
    )(rows_s, rows_d, w0b, pt, b0, w1, b1)


def _layer(ef, env, a1, a2, lb, lc):
    g = EPAD // BT
    return pl.pallas_call(
        _layer_body,
        grid=(g,),
        in_specs=[
            pl.BlockSpec((BT, DS), lambda i: (i, 0)),
            pl.BlockSpec((BT, DS), lambda i: (i, 0)),
            _full((DS, DS)),
            _full((DS, DS)),
            _full((DS, DS)),
            _full((DS, DS)),
        ],
        out_specs=pl.BlockSpec((BT, DS), lambda i: (i, 0)),
        out_shape=jax.ShapeDtypeStruct((EPAD, DS), jnp.float32),
    )(ef, env, a1, a2, lb, lc)


def _readout(ef, wr0, wr1):
    g = EPAD // BT
    return pl.pallas_call(
        _readout_body,
        grid=(g,),
        in_specs=[
            pl.BlockSpec((BT, DS), lambda i: (i, 0)),
            _full((DS, 8)),
            _full((8, 1)),
        ],
        out_specs=pl.BlockSpec((BT, 1), lambda i: (i, 0)),
        out_shape=jax.ShapeDtypeStruct((EPAD, 1), jnp.float32),
    )(ef, wr0, wr1)


def _combine(partials):
    p3 = partials.reshape(2, NE_PAD // 128, 128)
    return pl.pallas_call(
        _combine_body,
        grid=(1,),
        in_specs=[pl.BlockSpec((2, NE_PAD // 128, 128), lambda i: (0, 0, 0))],
        out_specs=pl.BlockSpec((NE_PAD // 128, 128), lambda i: (0, 0)),
        out_shape=jax.ShapeDtypeStruct((NE_PAD // 128, 128), jnp.float32),
    )(p3)


def kernel(positions, atom_types, edge_index, W_pair, W0, b0, W1, b1,
           L0a, L0b, L0c, L1a, L1b, L1c, Wr0, Wr1):
    src = edge_index[0]
    dst = edge_index[1]
    pad = EPAD - E
    srcp = jnp.concatenate([src, jnp.zeros((pad,), src.dtype)]).astype(jnp.int32)
    dstp = jnp.concatenate([dst, jnp.zeros((pad,), dst.dtype)]).astype(jnp.int32)
    src2 = srcp.reshape(ROWS, 128)
    dst2 = dstp.reshape(ROWS, 128)

    table = jnp.concatenate([positions, atom_types.astype(jnp.float32)[:, None],
                             jnp.zeros((N, 4), jnp.float32)], axis=1)

    both_idx = jnp.concatenate([src2, dst2], axis=0)
    rows_sd = _sc_gather(table, both_idx, 8, 16)
    rows_s = rows_sd[:ROWS].reshape(EPAD, 8)
    rows_d = rows_sd[ROWS:].reshape(EPAD, 8)

    w0b = W0[:NB]
    pt = W_pair @ W0[NB:]
    b0r = b0[None, :]
    b1r = b1[None, :]

    ef = _edge_mlp(rows_s, rows_d, w0b, pt, b0r, W1, b1r)

    zrows = jnp.zeros((SPAD_ROWS // _NS, DS), jnp.float32)
    for (Wa, Wb, Wc) in ((L0a, L0b, L0c), (L1a, L1b, L1c)):
        a1 = Wa[:DS]
        a2 = Wa[DS:] * (1.0 / AVG)
        node_env = _sc_scatter_env(ef.reshape(ROWS, 128, DS), dst2, zrows)
        env = _sc_gather(node_env, src2, DS, 8).reshape(EPAD, DS)
        ef = _layer(ef, env, a1, a2, Wb, Wc)

    e = _readout(ef, Wr0, Wr1)
    z1 = jnp.zeros((NE_PAD // _NS,), jnp.float32)
    partials = _sc_scatter_energy(e.reshape(ROWS, 128), dst2, z1)
    atom_energy = _combine(partials).reshape(NE_PAD)[:N]
    return atom_energy


# speedup vs baseline: 5.2176x; 1.2614x over previous
import functools
import math

import jax
import jax.numpy as jnp
from jax import lax
from jax.experimental import pallas as pl
from jax.experimental.pallas import tpu as pltpu
from jax.experimental.pallas import tpu_sc as plsc

N = 100000
E = 1600000
R_MAX = 4.0
NB = 8
NT = 4
DS = 32
AVG = 20.0

EPAD = 1638400
ROWS = EPAD // 128
BT = 4096

_NC = 2
_NS = 16
_NW = _NC * _NS

HALF = N // 2
SPAD_ROWS = 50048
NE_PAD = 100352


def _silu(x):
    return x / (1.0 + jnp.exp(-x))


def _sc_gather(table, idx2, d, chr_):
    rows = idx2.shape[0]
    rows_per_w = rows // _NW
    n_macro = rows_per_w // chr_
    mesh = plsc.VectorSubcoreMesh(core_axis_name="c", subcore_axis_name="s")

    def body(table_ref, idx_ref, out_ref, idx_v, rows_v, sem):
        wid = lax.axis_index("s") * _NC + lax.axis_index("c")
        base = wid * rows_per_w

        def macro(m, carry):
            rb = base + m * chr_
            pltpu.sync_copy(idx_ref.at[pl.ds(rb, chr_)], idx_v)
            handles = []
            for j in range(chr_):
                handles.append(pltpu.async_copy(table_ref.at[idx_v.at[j]], rows_v.at[j], sem))
            for h in handles:
                h.wait()
            pltpu.sync_copy(rows_v, out_ref.at[pl.ds(rb, chr_)])
            return carry

        lax.fori_loop(0, n_macro, macro, 0)

    f = pl.kernel(
        body,
        out_type=jax.ShapeDtypeStruct((rows, 128, d), jnp.float32),
        mesh=mesh,
        compiler_params=pltpu.CompilerParams(use_tc_tiling_on_sc=False),
        scratch_types=[
            pltpu.VMEM((chr_, 128), jnp.int32),
            pltpu.VMEM((chr_, 128, d), jnp.float32),
            pltpu.SemaphoreType.DMA,
        ],
    )
    return f(table, idx2)


def _sc_scatter_env(ef3, dst2, zrows):
    chr_ = 4
    rows_per_tile = ROWS // _NS
    n_macro = rows_per_tile // chr_
    z_per_tile = SPAD_ROWS // _NS
    o_per_tile = HALF // _NS
    mesh = plsc.VectorSubcoreMesh(core_axis_name="c", subcore_axis_name="s")

    def body(ef_ref, dst_ref, z_ref, out_ref, idx_v, ef_v, acc, sem):
        c = lax.axis_index("c")
        s = lax.axis_index("s")
        pltpu.sync_copy(z_ref.at[pl.ds(0, z_per_tile)], acc.at[pl.ds(s * z_per_tile, z_per_tile)])
        plsc.subcore_barrier()
        lo = c * HALF

        def macro(m, carry):
            rb = s * rows_per_tile + m * chr_
            pltpu.sync_copy(dst_ref.at[pl.ds(rb, chr_)], idx_v)
            pltpu.sync_copy(ef_ref.at[pl.ds(rb, chr_)], ef_v)
            for j in range(chr_):
                for k in range(8):
                    lid = idx_v[j, pl.ds(k * 16, 16)] - lo
                    ok = (lid >= 0) & (lid < HALF)
                    dummy = HALF + (k % 3) * 16 + lax.iota(jnp.int32, 16)
                    idx_v[j, pl.ds(k * 16, 16)] = jnp.where(ok, lid, dummy)
            handles = []
            for j in range(chr_):
                handles.append(pltpu.async_copy(ef_v.at[j], acc.at[idx_v.at[j]], sem, add=True))
            for h in handles:
                h.wait()
            return carry

        lax.fori_loop(0, n_macro, macro, 0)
        plsc.subcore_barrier()
        pltpu.sync_copy(acc.at[pl.ds(s * o_per_tile, o_per_tile)],
                        out_ref.at[pl.ds(c * HALF + s * o_per_tile, o_per_tile)])

    f = pl.kernel(
        body,
        out_type=jax.ShapeDtypeStruct((N, DS), jnp.float32),
        mesh=mesh,
        compiler_params=pltpu.CompilerParams(use_tc_tiling_on_sc=False),
        scratch_types=[
            pltpu.VMEM((chr_, 128), jnp.int32),
            pltpu.VMEM((chr_, 128, DS), jnp.float32),
            pltpu.VMEM_SHARED((SPAD_ROWS, DS), jnp.float32),
            pltpu.SemaphoreType.DMA,
        ],
    )
    return f(ef3, dst2, zrows)


def _sc_scatter_energy(e2, dst2, z1):
    chr_ = 16
    rows_per_tile = ROWS // _NW
    n_macro = rows_per_tile // chr_
    z_per_tile = NE_PAD // _NS
    mesh = plsc.VectorSubcoreMesh(core_axis_name="c", subcore_axis_name="s")

    def body(e_ref, dst_ref, z_ref, out_ref, idx_v, e_v, acc, sem):
        c = lax.axis_index("c")
        s = lax.axis_index("s")
        pltpu.sync_copy(z_ref, acc.at[pl.ds(s * z_per_tile, z_per_tile)])
        plsc.subcore_barrier()
        base = (c * _NS + s) * rows_per_tile

        def macro(m, carry):
            rb = base + m * chr_
            pltpu.sync_copy(dst_ref.at[pl.ds(rb, chr_)], idx_v)
            pltpu.sync_copy(e_ref.at[pl.ds(rb, chr_)], e_v)
            handles = []
            for j in range(chr_):
                handles.append(pltpu.async_copy(e_v.at[j], acc.at[idx_v.at[j]], sem, add=True))
            for h in handles:
                h.wait()
            return carry

        lax.fori_loop(0, n_macro, macro, 0)
        plsc.subcore_barrier()
        pltpu.sync_copy(acc.at[pl.ds(s * z_per_tile, z_per_tile)],
                        out_ref.at[c, pl.ds(s * z_per_tile, z_per_tile)])

    f = pl.kernel(
        body,
        out_type=jax.ShapeDtypeStruct((2, NE_PAD), jnp.float32),
        mesh=mesh,
        compiler_params=pltpu.CompilerParams(use_tc_tiling_on_sc=False),
        scratch_types=[
            pltpu.VMEM((chr_, 128), jnp.int32),
            pltpu.VMEM((chr_, 128), jnp.float32),
            pltpu.VMEM_SHARED((NE_PAD,), jnp.float32),
            pltpu.SemaphoreType.DMA,
        ],
    )
    return f(e2, dst2, z1)


def _edge_mlp_body(rs_ref, rd_ref, w0b_ref, pt_ref, b0_ref, w1_ref, b1_ref, out_ref):
    i = pl.program_id(0)
    rs = rs_ref[...]
    rd = rd_ref[...]
    d = rd - rs
    lane = jax.lax.broadcasted_iota(jnp.int32, (BT, 8), 1)
    d2 = jnp.where(lane < 3, d * d, 0.0)
    r2 = jnp.sum(d2, axis=1, keepdims=True)
    ts = jnp.sum(jnp.where(lane == 3, rs, 0.0), axis=1, keepdims=True)
    td = jnp.sum(jnp.where(lane == 3, rd, 0.0), axis=1, keepdims=True)
    pidf = ts * float(NT) + td
    r = jnp.sqrt(r2 + 1e-12)
    x = r * (1.0 / R_MAX)
    n = jax.lax.broadcasted_iota(jnp.int32, (BT, NB), 1).astype(jnp.float32) + 1.0
    arg = n * (jnp.pi * x)
    bes = math.sqrt(2.0 / R_MAX) * jnp.sin(arg) / (r + 1e-9)
    p = 6.0
    x2 = x * x
    x6 = x2 * x2 * x2
    cut = 1.0 - (p + 1.0) * (p + 2.0) / 2.0 * x6 + p * (p + 2.0) * x6 * x - p * (p + 1.0) / 2.0 * x6 * x2
    cut = jnp.where(x < 1.0, cut, 0.0)
    bes = bes * cut
    onehot = (jax.lax.broadcasted_iota(jnp.int32, (BT, NT * NT), 1).astype(jnp.float32) == pidf
              ).astype(jnp.float32)
    h = jnp.dot(bes, w0b_ref[...], preferred_element_type=jnp.float32)
    h = h + jnp.dot(onehot, pt_ref[...], preferred_element_type=jnp.float32)
    h = _silu(h + b0_ref[...])
    ef = _silu(jnp.dot(h, w1_ref[...], preferred_element_type=jnp.float32) + b1_ref[...])
    rows = i * BT + jax.lax.broadcasted_iota(jnp.int32, (BT, 1), 0)
    out_ref[...] = jnp.where(rows < E, ef, 0.0)


def _layer_body(ef_ref, env_ref, a1_ref, a2_ref, lb_ref, lc_ref, out_ref):
    i = pl.program_id(0)
    ef = ef_ref[...]
    z = _silu(jnp.dot(ef, a1_ref[...], preferred_element_type=jnp.float32)
              + jnp.dot(env_ref[...], a2_ref[...], preferred_element_type=jnp.float32))
    z = _silu(jnp.dot(z, lb_ref[...], preferred_element_type=jnp.float32))
    z = _silu(jnp.dot(z, lc_ref[...], preferred_element_type=jnp.float32))
    out = ef + z
    rows = i * BT + jax.lax.broadcasted_iota(jnp.int32, (BT, 1), 0)
    out_ref[...] = jnp.where(rows < E, out, 0.0)


def _layer2_readout_body(ef_ref, env_ref, a1_ref, a2_ref, lb_ref, lc_ref,
                         wr0_ref, wr1_ref, out_ref):
    i = pl.program_id(0)
    ef = ef_ref[...]
    z = _silu(jnp.dot(ef, a1_ref[...], preferred_element_type=jnp.float32)
              + jnp.dot(env_ref[...], a2_ref[...], preferred_element_type=jnp.float32))
    z = _silu(jnp.dot(z, lb_ref[...], preferred_element_type=jnp.float32))
    z = _silu(jnp.dot(z, lc_ref[...], preferred_element_type=jnp.float32))
    ef = ef + z
    s = _silu(jnp.dot(ef, wr0_ref[...], preferred_element_type=jnp.float32))
    e = jnp.dot(s, wr1_ref[...], preferred_element_type=jnp.float32)
    rows = i * BT + jax.lax.broadcasted_iota(jnp.int32, (BT, 1), 0)
    out_ref[...] = jnp.where(rows < E, e, 0.0)


def _readout_body(ef_ref, wr0_ref, wr1_ref, out_ref):
    i = pl.program_id(0)
    s = _silu(jnp.dot(ef_ref[...], wr0_ref[...], preferred_element_type=jnp.float32))
    e = jnp.dot(s, wr1_ref[...], preferred_element_type=jnp.float32)
    rows = i * BT + jax.lax.broadcasted_iota(jnp.int32, (BT, 1), 0)
    out_ref[...] = jnp.where(rows < E, e, 0.0)


def _combine_body(p_ref, out_ref):
    out_ref[...] = (p_ref[0] + p_ref[1]) * (1.0 / math.sqrt(AVG))


def _full(shape):
    return pl.BlockSpec(shape, lambda i: (0, 0))


def _edge_mlp(rows_s, rows_d, w0b, pt, b0, w1, b1):
    g = EPAD // BT
    return pl.pallas_call(
        _edge_mlp_body,
        grid=(g,),
        in_specs=[
            pl.BlockSpec((BT, 8), lambda i: (i, 0)),
            pl.BlockSpec((BT, 8), lambda i: (i, 0)),
            _full((NB, DS)),
            _full((NT * NT, DS)),
            _full((1, DS)),
            _full((DS, DS)),
            _full((1, DS)),
        ],
        out_specs=pl.BlockSpec((BT, DS), lambda i: (i, 0)),
        out_shape=jax.ShapeDtypeStruct((EPAD, DS), jnp.float32),
    )(rows_s, rows_d, w0b, pt, b0, w1, b1)


def _layer(ef, env, a1, a2, lb, lc):
    g = EPAD // BT
    return pl.pallas_call(
        _layer_body,
        grid=(g,),
        in_specs=[
            pl.BlockSpec((BT, DS), lambda i: (i, 0)),
            pl.BlockSpec((BT, DS), lambda i: (i, 0)),
            _full((DS, DS)),
            _full((DS, DS)),
            _full((DS, DS)),
            _full((DS, DS)),
        ],
        out_specs=pl.BlockSpec((BT, DS), lambda i: (i, 0)),
        out_shape=jax.ShapeDtypeStruct((EPAD, DS), jnp.float32),
    )(ef, env, a1, a2, lb, lc)


def _layer2_readout(ef, env, a1, a2, lb, lc, wr0, wr1):
    g = EPAD // BT
    return pl.pallas_call(
        _layer2_readout_body,
        grid=(g,),
        in_specs=[
            pl.BlockSpec((BT, DS), lambda i: (i, 0)),
            pl.BlockSpec((BT, DS), lambda i: (i, 0)),
            _full((DS, DS)),
            _full((DS, DS)),
            _full((DS, DS)),
            _full((DS, DS)),
            _full((DS, 8)),
            _full((8, 1)),
        ],
        out_specs=pl.BlockSpec((BT, 1), lambda i: (i, 0)),
        out_shape=jax.ShapeDtypeStruct((EPAD, 1), jnp.float32),
    )(ef, env, a1, a2, lb, lc, wr0, wr1)


def _readout(ef, wr0, wr1):
    g = EPAD // BT
    return pl.pallas_call(
        _readout_body,
        grid=(g,),
        in_specs=[
            pl.BlockSpec((BT, DS), lambda i: (i, 0)),
            _full((DS, 8)),
            _full((8, 1)),
        ],
        out_specs=pl.BlockSpec((BT, 1), lambda i: (i, 0)),
        out_shape=jax.ShapeDtypeStruct((EPAD, 1), jnp.float32),
    )(ef, wr0, wr1)


def _combine(partials):
    p3 = partials.reshape(2, NE_PAD // 128, 128)
    return pl.pallas_call(
        _combine_body,
        grid=(1,),
        in_specs=[pl.BlockSpec((2, NE_PAD // 128, 128), lambda i: (0, 0, 0))],
        out_specs=pl.BlockSpec((NE_PAD // 128, 128), lambda i: (0, 0)),
        out_shape=jax.ShapeDtypeStruct((NE_PAD // 128, 128), jnp.float32),
    )(p3)


def kernel(positions, atom_types, edge_index, W_pair, W0, b0, W1, b1,
           L0a, L0b, L0c, L1a, L1b, L1c, Wr0, Wr1):
    src = edge_index[0]
    dst = edge_index[1]
    pad = EPAD - E
    srcp = jnp.concatenate([src, jnp.zeros((pad,), src.dtype)]).astype(jnp.int32)
    dstp = jnp.concatenate([dst, jnp.zeros((pad,), dst.dtype)]).astype(jnp.int32)
    src2 = srcp.reshape(ROWS, 128)
    dst2 = dstp.reshape(ROWS, 128)

    table = jnp.concatenate([positions, atom_types.astype(jnp.float32)[:, None],
                             jnp.zeros((N, 4), jnp.float32)], axis=1)

    both_idx = jnp.concatenate([src2, dst2], axis=0)
    rows_sd = _sc_gather(table, both_idx, 8, 16)
    rows_s = rows_sd[:ROWS].reshape(EPAD, 8)
    rows_d = rows_sd[ROWS:].reshape(EPAD, 8)

    w0b = W0[:NB]
    pt = W_pair @ W0[NB:]
    b0r = b0[None, :]
    b1r = b1[None, :]

    ef = _edge_mlp(rows_s, rows_d, w0b, pt, b0r, W1, b1r)

    zrows = jnp.zeros((SPAD_ROWS // _NS, DS), jnp.float32)

    node_env = _sc_scatter_env(ef.reshape(ROWS, 128, DS), dst2, zrows)
    env = _sc_gather(node_env, src2, DS, 16).reshape(EPAD, DS)
    ef = _layer(ef, env, L0a[:DS], L0a[DS:] * (1.0 / AVG), L0b, L0c)

    node_env = _sc_scatter_env(ef.reshape(ROWS, 128, DS), dst2, zrows)
    env = _sc_gather(node_env, src2, DS, 16).reshape(EPAD, DS)
    e = _layer2_readout(ef, env, L1a[:DS], L1a[DS:] * (1.0 / AVG), L1b, L1c, Wr0, Wr1)
    z1 = jnp.zeros((NE_PAD // _NS,), jnp.float32)
    partials = _sc_scatter_energy(e.reshape(ROWS, 128), dst2, z1)
    atom_energy = _combine(partials).reshape(NE_PAD)[:N]
    return atom_energy


# speedup vs baseline: 6.8405x; 1.3111x over previous
import functools
import math

import jax
import jax.numpy as jnp
from jax import lax
from jax.experimental import pallas as pl
from jax.experimental.pallas import tpu as pltpu
from jax.experimental.pallas import tpu_sc as plsc

N = 100000
E = 1600000
R_MAX = 4.0
NB = 8
NT = 4
DS = 32
AVG = 20.0

EPAD = 1638400
ROWS = EPAD // 128
BT = 4096

_NC = 2
_NS = 16
_NW = _NC * _NS

HALF = N // 2
SPAD_ROWS = 50048
NE_PAD = 100352


def _silu(x):
    return x / (1.0 + jnp.exp(-x))


def _sc_gather(table, idx2, d, chr_):
    rows = idx2.shape[0]
    rows_per_w = rows // _NW
    n_macro = rows_per_w // chr_
    mesh = plsc.VectorSubcoreMesh(core_axis_name="c", subcore_axis_name="s")

    def body(table_ref, idx_ref, out_ref, idx_v, rows_v, sem):
        wid = lax.axis_index("s") * _NC + lax.axis_index("c")
        base = wid * rows_per_w

        def macro(m, carry):
            rb = base + m * chr_
            pltpu.sync_copy(idx_ref.at[pl.ds(rb, chr_)], idx_v)
            handles = []
            for j in range(chr_):
                handles.append(pltpu.async_copy(table_ref.at[idx_v.at[j]], rows_v.at[j], sem))
            for h in handles:
                h.wait()
            pltpu.sync_copy(rows_v, out_ref.at[pl.ds(rb, chr_)])
            return carry

        lax.fori_loop(0, n_macro, macro, 0)

    f = pl.kernel(
        body,
        out_type=jax.ShapeDtypeStruct((rows, 128, d), jnp.float32),
        mesh=mesh,
        compiler_params=pltpu.CompilerParams(use_tc_tiling_on_sc=False),
        scratch_types=[
            pltpu.VMEM((chr_, 128), jnp.int32),
            pltpu.VMEM((chr_, 128, d), jnp.float32),
            pltpu.SemaphoreType.DMA,
        ],
    )
    return f(table, idx2)


def _sc_scatter_env(ef3, dst2, zrows):
    chr_ = 4
    rows_per_tile = ROWS // _NS
    n_macro = rows_per_tile // chr_
    z_per_tile = SPAD_ROWS // _NS
    o_per_tile = HALF // _NS
    mesh = plsc.VectorSubcoreMesh(core_axis_name="c", subcore_axis_name="s")

    def body(ef_ref, dst_ref, z_ref, out_ref, idx_v, ef_v, acc, sem):
        c = lax.axis_index("c")
        s = lax.axis_index("s")
        pltpu.sync_copy(z_ref.at[pl.ds(0, z_per_tile)], acc.at[pl.ds(s * z_per_tile, z_per_tile)])
        plsc.subcore_barrier()
        lo = c * HALF

        def macro(m, carry):
            rb = s * rows_per_tile + m * chr_
            pltpu.sync_copy(dst_ref.at[pl.ds(rb, chr_)], idx_v)
            pltpu.sync_copy(ef_ref.at[pl.ds(rb, chr_)], ef_v)
            for j in range(chr_):
                for k in range(8):
                    lid = idx_v[j, pl.ds(k * 16, 16)] - lo
                    ok = (lid >= 0) & (lid < HALF)
                    dummy = HALF + (k % 3) * 16 + lax.iota(jnp.int32, 16)
                    idx_v[j, pl.ds(k * 16, 16)] = jnp.where(ok, lid, dummy)
            handles = []
            for j in range(chr_):
                handles.append(pltpu.async_copy(ef_v.at[j], acc.at[idx_v.at[j]], sem, add=True))
            for h in handles:
                h.wait()
            return carry

        lax.fori_loop(0, n_macro, macro, 0)
        plsc.subcore_barrier()
        pltpu.sync_copy(acc.at[pl.ds(s * o_per_tile, o_per_tile)],
                        out_ref.at[pl.ds(c * HALF + s * o_per_tile, o_per_tile)])

    f = pl.kernel(
        body,
        out_type=jax.ShapeDtypeStruct((N, DS), jnp.float32),
        mesh=mesh,
        compiler_params=pltpu.CompilerParams(use_tc_tiling_on_sc=False),
        scratch_types=[
            pltpu.VMEM((chr_, 128), jnp.int32),
            pltpu.VMEM((chr_, 128, DS), jnp.float32),
            pltpu.VMEM_SHARED((SPAD_ROWS, DS), jnp.float32),
            pltpu.SemaphoreType.DMA,
        ],
    )
    return f(ef3, dst2, zrows)


def _sc_scatter_energy(e2, dst2, z1):
    chr_ = 16
    rows_per_tile = ROWS // _NW
    n_macro = rows_per_tile // chr_
    z_per_tile = NE_PAD // _NS
    mesh = plsc.VectorSubcoreMesh(core_axis_name="c", subcore_axis_name="s")

    def body(e_ref, dst_ref, z_ref, out_ref, idx_v, e_v, acc, sem):
        c = lax.axis_index("c")
        s = lax.axis_index("s")
        pltpu.sync_copy(z_ref, acc.at[pl.ds(s * z_per_tile, z_per_tile)])
        plsc.subcore_barrier()
        base = (c * _NS + s) * rows_per_tile

        def macro(m, carry):
            rb = base + m * chr_
            pltpu.sync_copy(dst_ref.at[pl.ds(rb, chr_)], idx_v)
            pltpu.sync_copy(e_ref.at[pl.ds(rb, chr_)], e_v)
            handles = []
            for j in range(chr_):
                handles.append(pltpu.async_copy(e_v.at[j], acc.at[idx_v.at[j]], sem, add=True))
            for h in handles:
                h.wait()
            return carry

        lax.fori_loop(0, n_macro, macro, 0)
        plsc.subcore_barrier()
        pltpu.sync_copy(acc.at[pl.ds(s * z_per_tile, z_per_tile)],
                        out_ref.at[c, pl.ds(s * z_per_tile, z_per_tile)])

    f = pl.kernel(
        body,
        out_type=jax.ShapeDtypeStruct((2, NE_PAD), jnp.float32),
        mesh=mesh,
        compiler_params=pltpu.CompilerParams(use_tc_tiling_on_sc=False),
        scratch_types=[
            pltpu.VMEM((chr_, 128), jnp.int32),
            pltpu.VMEM((chr_, 128), jnp.float32),
            pltpu.VMEM_SHARED((NE_PAD,), jnp.float32),
            pltpu.SemaphoreType.DMA,
        ],
    )
    return f(e2, dst2, z1)


def _edge_mlp_body(rs_ref, rd_ref, w0bt_ref, ptt_ref, b0t_ref, w1t_ref, b1t_ref, out_ref):
    i = pl.program_id(0)
    rst = rs_ref[...].T
    rdt = rd_ref[...].T
    d = rdt - rst
    d2 = d * d
    r2 = d2[0:1] + d2[1:2] + d2[2:3]
    pidf = rst[3:4] * float(NT) + rdt[3:4]
    r = jnp.sqrt(r2 + 1e-12)
    x = r * (1.0 / R_MAX)
    xc = jnp.minimum(x, 1.0)
    t = xc - 0.5
    u2 = (math.pi * math.pi) * (t * t)
    s1m = 1.0 + u2 * (-1.0 / 2 + u2 * (1.0 / 24 + u2 * (-1.0 / 720 + u2 * (1.0 / 40320 + u2 * (-1.0 / 3628800)))))
    c1m = -(math.pi * t) * (1.0 + u2 * (-1.0 / 6 + u2 * (1.0 / 120 + u2 * (-1.0 / 5040 + u2 * (1.0 / 362880 + u2 * (-1.0 / 39916800))))))
    v2 = (math.pi * math.pi) * (xc * xc)
    s1d = (math.pi * xc) * (1.0 + v2 * (-1.0 / 6 + v2 * (1.0 / 120 + v2 * (-1.0 / 5040))))
    c1d = 1.0 + v2 * (-1.0 / 2 + v2 * (1.0 / 24 + v2 * (-1.0 / 720 + v2 * (1.0 / 40320))))
    small = xc < 0.25
    s1 = jnp.where(small, s1d, s1m)
    c1 = jnp.where(small, c1d, c1m)
    tc1 = c1 + c1
    s2 = tc1 * s1
    s3 = tc1 * s2 - s1
    s4 = tc1 * s3 - s2
    s5 = tc1 * s4 - s3
    s6 = tc1 * s5 - s4
    s7 = tc1 * s6 - s5
    s8 = tc1 * s7 - s6
    bes = jnp.concatenate([s1, s2, s3, s4, s5, s6, s7, s8], axis=0)
    p = 6.0
    x2 = x * x
    x6 = x2 * x2 * x2
    cut = 1.0 - (p + 1.0) * (p + 2.0) / 2.0 * x6 + p * (p + 2.0) * x6 * x - p * (p + 1.0) / 2.0 * x6 * x2
    cut = jnp.where(x < 1.0, cut, 0.0)
    pref = (math.sqrt(2.0 / R_MAX) * cut) / (r + 1e-9)
    bes = bes * pref
    onehot = (jax.lax.broadcasted_iota(jnp.int32, (NT * NT, BT), 0).astype(jnp.float32) == pidf
              ).astype(jnp.float32)
    h = jnp.dot(w0bt_ref[...], bes, preferred_element_type=jnp.float32)
    h = h + jnp.dot(ptt_ref[...], onehot, preferred_element_type=jnp.float32)
    h = _silu(h + b0t_ref[...])
    eft = _silu(jnp.dot(w1t_ref[...], h, preferred_element_type=jnp.float32) + b1t_ref[...])
    cols = jax.lax.broadcasted_iota(jnp.int32, (1, BT), 1)
    eft = jnp.where(cols < (E - i * BT), eft, 0.0)
    out_ref[...] = eft.T


def _layer_body(ef_ref, env_ref, a1_ref, a2_ref, lb_ref, lc_ref, out_ref):
    i = pl.program_id(0)
    ef = ef_ref[...]
    z = _silu(jnp.dot(ef, a1_ref[...], preferred_element_type=jnp.float32)
              + jnp.dot(env_ref[...], a2_ref[...], preferred_element_type=jnp.float32))
    z = _silu(jnp.dot(z, lb_ref[...], preferred_element_type=jnp.float32))
    z = _silu(jnp.dot(z, lc_ref[...], preferred_element_type=jnp.float32))
    out = ef + z
    rows = jax.lax.broadcasted_iota(jnp.int32, (BT, DS), 0)
    out_ref[...] = jnp.where(rows < (E - i * BT), out, 0.0)


def _layer2_readout_body(ef_ref, env_ref, a1_ref, a2_ref, lb_ref, lc_ref,
                         wr0_ref, wr1_ref, out_ref):
    i = pl.program_id(0)
    ef = ef_ref[...]
    z = _silu(jnp.dot(ef, a1_ref[...], preferred_element_type=jnp.float32)
              + jnp.dot(env_ref[...], a2_ref[...], preferred_element_type=jnp.float32))
    z = _silu(jnp.dot(z, lb_ref[...], preferred_element_type=jnp.float32))
    z = _silu(jnp.dot(z, lc_ref[...], preferred_element_type=jnp.float32))
    ef = ef + z
    s = _silu(jnp.dot(ef, wr0_ref[...], preferred_element_type=jnp.float32))
    e = jnp.dot(s, wr1_ref[...], preferred_element_type=jnp.float32)
    rows = jax.lax.broadcasted_iota(jnp.int32, (BT, 1), 0)
    out_ref[...] = jnp.where(rows < (E - i * BT), e, 0.0)


def _readout_body(ef_ref, wr0_ref, wr1_ref, out_ref):
    i = pl.program_id(0)
    s = _silu(jnp.dot(ef_ref[...], wr0_ref[...], preferred_element_type=jnp.float32))
    e = jnp.dot(s, wr1_ref[...], preferred_element_type=jnp.float32)
    rows = i * BT + jax.lax.broadcasted_iota(jnp.int32, (BT, 1), 0)
    out_ref[...] = jnp.where(rows < E, e, 0.0)


def _combine_body(p_ref, out_ref):
    out_ref[...] = (p_ref[0] + p_ref[1]) * (1.0 / math.sqrt(AVG))


def _full(shape):
    return pl.BlockSpec(shape, lambda i: (0, 0))


def _edge_mlp(rows_s, rows_d, w0b, pt, b0, w1, b1):
    g = EPAD // BT
    return pl.pallas_call(
        _edge_mlp_body,
        grid=(g,),
        in_specs=[
            pl.BlockSpec((BT, 8), lambda i: (i, 0)),
            pl.BlockSpec((BT, 8), lambda i: (i, 0)),
            _full((DS, NB)),
            _full((DS, NT * NT)),
            _full((DS, 1)),
            _full((DS, DS)),
            _full((DS, 1)),
        ],
        out_specs=pl.BlockSpec((BT, DS), lambda i: (i, 0)),
        out_shape=jax.ShapeDtypeStruct((EPAD, DS), jnp.float32),
    )(rows_s, rows_d, w0b, pt, b0, w1, b1)


def _layer(ef, env, a1, a2, lb, lc):
    g = EPAD // BT
    return pl.pallas_call(
        _layer_body,
        grid=(g,),
        in_specs=[
            pl.BlockSpec((BT, DS), lambda i: (i, 0)),
            pl.BlockSpec((BT, DS), lambda i: (i, 0)),
            _full((DS, DS)),
            _full((DS, DS)),
            _full((DS, DS)),
            _full((DS, DS)),
        ],
        out_specs=pl.BlockSpec((BT, DS), lambda i: (i, 0)),
        out_shape=jax.ShapeDtypeStruct((EPAD, DS), jnp.float32),
    )(ef, env, a1, a2, lb, lc)


def _layer2_readout(ef, env, a1, a2, lb, lc, wr0, wr1):
    g = EPAD // BT
    return pl.pallas_call(
        _layer2_readout_body,
        grid=(g,),
        in_specs=[
            pl.BlockSpec((BT, DS), lambda i: (i, 0)),
            pl.BlockSpec((BT, DS), lambda i: (i, 0)),
            _full((DS, DS)),
            _full((DS, DS)),
            _full((DS, DS)),
            _full((DS, DS)),
            _full((DS, 8)),
            _full((8, 1)),
        ],
        out_specs=pl.BlockSpec((BT, 1), lambda i: (i, 0)),
        out_shape=jax.ShapeDtypeStruct((EPAD, 1), jnp.float32),
    )(ef, env, a1, a2, lb, lc, wr0, wr1)


def _readout(ef, wr0, wr1):
    g = EPAD // BT
    return pl.pallas_call(
        _readout_body,
        grid=(g,),
        in_specs=[
            pl.BlockSpec((BT, DS), lambda i: (i, 0)),
            _full((DS, 8)),
            _full((8, 1)),
        ],
        out_specs=pl.BlockSpec((BT, 1), lambda i: (i, 0)),
        out_shape=jax.ShapeDtypeStruct((EPAD, 1), jnp.float32),
    )(ef, wr0, wr1)


def _combine(partials):
    p3 = partials.reshape(2, NE_PAD // 128, 128)
    return pl.pallas_call(
        _combine_body,
        grid=(1,),
        in_specs=[pl.BlockSpec((2, NE_PAD // 128, 128), lambda i: (0, 0, 0))],
        out_specs=pl.BlockSpec((NE_PAD // 128, 128), lambda i: (0, 0)),
        out_shape=jax.ShapeDtypeStruct((NE_PAD // 128, 128), jnp.float32),
    )(p3)


def kernel(positions, atom_types, edge_index, W_pair, W0, b0, W1, b1,
           L0a, L0b, L0c, L1a, L1b, L1c, Wr0, Wr1):
    src = edge_index[0]
    dst = edge_index[1]
    pad = EPAD - E
    srcp = jnp.concatenate([src, jnp.zeros((pad,), src.dtype)]).astype(jnp.int32)
    dstp = jnp.concatenate([dst, jnp.zeros((pad,), dst.dtype)]).astype(jnp.int32)
    src2 = srcp.reshape(ROWS, 128)
    dst2 = dstp.reshape(ROWS, 128)

    table = jnp.concatenate([positions, atom_types.astype(jnp.float32)[:, None],
                             jnp.zeros((N, 4), jnp.float32)], axis=1)

    both_idx = jnp.concatenate([src2, dst2], axis=0)
    rows_sd = _sc_gather(table, both_idx, 8, 16)
    rows_s = rows_sd[:ROWS].reshape(EPAD, 8)
    rows_d = rows_sd[ROWS:].reshape(EPAD, 8)

    w0bt = W0[:NB].T
    ptt = (W_pair @ W0[NB:]).T
    b0t = b0[:, None]
    b1t = b1[:, None]

    ef = _edge_mlp(rows_s, rows_d, w0bt, ptt, b0t, W1.T, b1t)

    zrows = jnp.zeros((SPAD_ROWS // _NS, DS), jnp.float32)

    node_env = _sc_scatter_env(ef.reshape(ROWS, 128, DS), dst2, zrows)
    env = _sc_gather(node_env, src2, DS, 16).reshape(EPAD, DS)
    ef = _layer(ef, env, L0a[:DS], L0a[DS:] * (1.0 / AVG), L0b, L0c)

    node_env = _sc_scatter_env(ef.reshape(ROWS, 128, DS), dst2, zrows)
    env = _sc_gather(node_env, src2, DS, 16).reshape(EPAD, DS)
    e = _layer2_readout(ef, env, L1a[:DS], L1a[DS:] * (1.0 / AVG), L1b, L1c, Wr0, Wr1)
    z1 = jnp.zeros((NE_PAD // _NS,), jnp.float32)
    partials = _sc_scatter_energy(e.reshape(ROWS, 128), dst2, z1)
    atom_energy = _combine(partials).reshape(NE_PAD)[:N]
    return atom_energy


# speedup vs baseline: 7.1284x; 1.0421x over previous
import functools
import math

import jax
import jax.numpy as jnp
from jax import lax
from jax.experimental import pallas as pl
from jax.experimental.pallas import tpu as pltpu
from jax.experimental.pallas import tpu_sc as plsc

N = 100000
E = 1600000
R_MAX = 4.0
NB = 8
NT = 4
DS = 32
AVG = 20.0

EPAD = 1638400
ROWS = EPAD // 128
BT = 4096

_NC = 2
_NS = 16
_NW = _NC * _NS

HALF = N // 2
SPAD_ROWS = 50048
NE_PAD = 100352


def _silu(x):
    return x / (1.0 + jnp.exp(-x))


def _sc_gather(table, idx2, d, chr_):
    rows = idx2.shape[0]
    rows_per_w = rows // _NW
    n_macro = rows_per_w // chr_
    mesh = plsc.VectorSubcoreMesh(core_axis_name="c", subcore_axis_name="s")

    def body(table_ref, idx_ref, out_ref, idx_v, rows_v, sem):
        wid = lax.axis_index("s") * _NC + lax.axis_index("c")
        base = wid * rows_per_w

        def macro(m, carry):
            rb = base + m * chr_
            pltpu.sync_copy(idx_ref.at[pl.ds(rb, chr_)], idx_v)
            handles = []
            for j in range(chr_):
                handles.append(pltpu.async_copy(table_ref.at[idx_v.at[j]], rows_v.at[j], sem))
            for h in handles:
                h.wait()
            pltpu.sync_copy(rows_v, out_ref.at[pl.ds(rb, chr_)])
            return carry

        lax.fori_loop(0, n_macro, macro, 0)

    f = pl.kernel(
        body,
        out_type=jax.ShapeDtypeStruct((rows, 128, d), jnp.float32),
        mesh=mesh,
        compiler_params=pltpu.CompilerParams(use_tc_tiling_on_sc=False),
        scratch_types=[
            pltpu.VMEM((chr_, 128), jnp.int32),
            pltpu.VMEM((chr_, 128, d), jnp.float32),
            pltpu.SemaphoreType.DMA,
        ],
    )
    return f(table, idx2)


def _sc_scatter_env(ef3, dst2, zrows):
    chr_ = 4
    rows_per_tile = ROWS // _NS
    n_macro = rows_per_tile // chr_
    z_per_tile = SPAD_ROWS // _NS
    o_per_tile = HALF // _NS
    mesh = plsc.VectorSubcoreMesh(core_axis_name="c", subcore_axis_name="s")

    def body(ef_ref, dst_ref, z_ref, out_ref, idx_v, ef_v, acc, sem):
        c = lax.axis_index("c")
        s = lax.axis_index("s")
        pltpu.sync_copy(z_ref.at[pl.ds(0, z_per_tile)], acc.at[pl.ds(s * z_per_tile, z_per_tile)])
        plsc.subcore_barrier()
        lo = c * HALF

        def macro(m, carry):
            rb = s * rows_per_tile + m * chr_
            pltpu.sync_copy(dst_ref.at[pl.ds(rb, chr_)], idx_v)
            pltpu.sync_copy(ef_ref.at[pl.ds(rb, chr_)], ef_v)
            for j in range(chr_):
                for k in range(8):
                    lid = idx_v[j, pl.ds(k * 16, 16)] - lo
                    ok = (lid >= 0) & (lid < HALF)
                    dummy = HALF + (k % 3) * 16 + lax.iota(jnp.int32, 16)
                    idx_v[j, pl.ds(k * 16, 16)] = jnp.where(ok, lid, dummy)
            handles = []
            for j in range(chr_):
                handles.append(pltpu.async_copy(ef_v.at[j], acc.at[idx_v.at[j]], sem, add=True))
            for h in handles:
                h.wait()
            return carry

        lax.fori_loop(0, n_macro, macro, 0)
        plsc.subcore_barrier()
        pltpu.sync_copy(acc.at[pl.ds(s * o_per_tile, o_per_tile)],
                        out_ref.at[pl.ds(c * HALF + s * o_per_tile, o_per_tile)])

    f = pl.kernel(
        body,
        out_type=jax.ShapeDtypeStruct((N, DS), jnp.float32),
        mesh=mesh,
        compiler_params=pltpu.CompilerParams(use_tc_tiling_on_sc=False),
        scratch_types=[
            pltpu.VMEM((chr_, 128), jnp.int32),
            pltpu.VMEM((chr_, 128, DS), jnp.float32),
            pltpu.VMEM_SHARED((SPAD_ROWS, DS), jnp.float32),
            pltpu.SemaphoreType.DMA,
        ],
    )
    return f(ef3, dst2, zrows)


def _sc_scatter_energy(e2, dst2, z1):
    chr_ = 16
    rows_per_tile = ROWS // _NW
    n_macro = rows_per_tile // chr_
    z_per_tile = NE_PAD // _NS
    mesh = plsc.VectorSubcoreMesh(core_axis_name="c", subcore_axis_name="s")

    def body(e_ref, dst_ref, z_ref, out_ref, idx_v, e_v, acc, sem):
        c = lax.axis_index("c")
        s = lax.axis_index("s")
        pltpu.sync_copy(z_ref, acc.at[pl.ds(s * z_per_tile, z_per_tile)])
        plsc.subcore_barrier()
        base = (c * _NS + s) * rows_per_tile

        def macro(m, carry):
            rb = base + m * chr_
            pltpu.sync_copy(dst_ref.at[pl.ds(rb, chr_)], idx_v)
            pltpu.sync_copy(e_ref.at[pl.ds(rb, chr_)], e_v)
            handles = []
            for j in range(chr_):
                handles.append(pltpu.async_copy(e_v.at[j], acc.at[idx_v.at[j]], sem, add=True))
            for h in handles:
                h.wait()
            return carry

        lax.fori_loop(0, n_macro, macro, 0)
        plsc.subcore_barrier()
        pltpu.sync_copy(acc.at[pl.ds(s * z_per_tile, z_per_tile)],
                        out_ref.at[c, pl.ds(s * z_per_tile, z_per_tile)])

    f = pl.kernel(
        body,
        out_type=jax.ShapeDtypeStruct((2, NE_PAD), jnp.float32),
        mesh=mesh,
        compiler_params=pltpu.CompilerParams(use_tc_tiling_on_sc=False),
        scratch_types=[
            pltpu.VMEM((chr_, 128), jnp.int32),
            pltpu.VMEM((chr_, 128), jnp.float32),
            pltpu.VMEM_SHARED((NE_PAD,), jnp.float32),
            pltpu.SemaphoreType.DMA,
        ],
    )
    return f(e2, dst2, z1)


def _edge_mlp_body(rs_ref, rd_ref, w0bt_ref, ptt_ref, b0t_ref, w1t_ref, b1t_ref, out_ref):
    i = pl.program_id(0)
    rst = rs_ref[...].T
    rdt = rd_ref[...].T
    d = rdt - rst
    d2 = d * d
    r2 = d2[0:1] + d2[1:2] + d2[2:3]
    pidf = rst[3:4] * float(NT) + rdt[3:4]
    r = jnp.sqrt(r2 + 1e-12)
    x = r * (1.0 / R_MAX)
    xc = jnp.minimum(x, 1.0)
    t = xc - 0.5
    u2 = (math.pi * math.pi) * (t * t)
    s1m = 1.0 + u2 * (-1.0 / 2 + u2 * (1.0 / 24 + u2 * (-1.0 / 720 + u2 * (1.0 / 40320 + u2 * (-1.0 / 3628800)))))
    c1m = -(math.pi * t) * (1.0 + u2 * (-1.0 / 6 + u2 * (1.0 / 120 + u2 * (-1.0 / 5040 + u2 * (1.0 / 362880 + u2 * (-1.0 / 39916800))))))
    v2 = (math.pi * math.pi) * (xc * xc)
    s1d = (math.pi * xc) * (1.0 + v2 * (-1.0 / 6 + v2 * (1.0 / 120 + v2 * (-1.0 / 5040))))
    c1d = 1.0 + v2 * (-1.0 / 2 + v2 * (1.0 / 24 + v2 * (-1.0 / 720 + v2 * (1.0 / 40320))))
    small = xc < 0.25
    s1 = jnp.where(small, s1d, s1m)
    c1 = jnp.where(small, c1d, c1m)
    tc1 = c1 + c1
    s2 = tc1 * s1
    s3 = tc1 * s2 - s1
    s4 = tc1 * s3 - s2
    s5 = tc1 * s4 - s3
    s6 = tc1 * s5 - s4
    s7 = tc1 * s6 - s5
    s8 = tc1 * s7 - s6
    bes = jnp.concatenate([s1, s2, s3, s4, s5, s6, s7, s8], axis=0)
    p = 6.0
    x2 = x * x
    x6 = x2 * x2 * x2
    cut = 1.0 - (p + 1.0) * (p + 2.0) / 2.0 * x6 + p * (p + 2.0) * x6 * x - p * (p + 1.0) / 2.0 * x6 * x2
    cut = jnp.where(x < 1.0, cut, 0.0)
    pref = (math.sqrt(2.0 / R_MAX) * cut) / (r + 1e-9)
    bes = bes * pref
    onehot = (jax.lax.broadcasted_iota(jnp.int32, (NT * NT, BT), 0).astype(jnp.float32) == pidf
              ).astype(jnp.float32)
    h = jnp.dot(w0bt_ref[...], bes, preferred_element_type=jnp.float32)
    h = h + jnp.dot(ptt_ref[...], onehot, preferred_element_type=jnp.float32)
    h = _silu(h + b0t_ref[...])
    eft = _silu(jnp.dot(w1t_ref[...], h, preferred_element_type=jnp.float32) + b1t_ref[...])
    cols = jax.lax.broadcasted_iota(jnp.int32, (1, BT), 1)
    eft = jnp.where(cols < (E - i * BT), eft, 0.0)
    out_ref[...] = eft.T


def _layer_body(ef_ref, env_ref, a1_ref, a2_ref, lb_ref, lc_ref, out_ref):
    i = pl.program_id(0)
    eft = ef_ref[...].T
    envt = env_ref[...].T
    z = _silu(jnp.dot(a1_ref[...], eft, preferred_element_type=jnp.float32)
              + jnp.dot(a2_ref[...], envt, preferred_element_type=jnp.float32))
    z = _silu(jnp.dot(lb_ref[...], z, preferred_element_type=jnp.float32))
    z = _silu(jnp.dot(lc_ref[...], z, preferred_element_type=jnp.float32))
    out = eft + z
    cols = jax.lax.broadcasted_iota(jnp.int32, (1, BT), 1)
    out = jnp.where(cols < (E - i * BT), out, 0.0)
    out_ref[...] = out.T


def _layer2_readout_body(ef_ref, env_ref, a1_ref, a2_ref, lb_ref, lc_ref,
                         wr0_ref, wr1_ref, out_ref):
    i = pl.program_id(0)
    eft = ef_ref[...].T
    envt = env_ref[...].T
    z = _silu(jnp.dot(a1_ref[...], eft, preferred_element_type=jnp.float32)
              + jnp.dot(a2_ref[...], envt, preferred_element_type=jnp.float32))
    z = _silu(jnp.dot(lb_ref[...], z, preferred_element_type=jnp.float32))
    z = _silu(jnp.dot(lc_ref[...], z, preferred_element_type=jnp.float32))
    eft = eft + z
    s = _silu(jnp.dot(wr0_ref[...], eft, preferred_element_type=jnp.float32))
    e = jnp.dot(wr1_ref[...], s, preferred_element_type=jnp.float32)
    cols = jax.lax.broadcasted_iota(jnp.int32, (1, BT), 1)
    e = jnp.where(cols < (E - i * BT), e, 0.0)
    out_ref[...] = e.T


def _readout_body(ef_ref, wr0_ref, wr1_ref, out_ref):
    i = pl.program_id(0)
    s = _silu(jnp.dot(ef_ref[...], wr0_ref[...], preferred_element_type=jnp.float32))
    e = jnp.dot(s, wr1_ref[...], preferred_element_type=jnp.float32)
    rows = i * BT + jax.lax.broadcasted_iota(jnp.int32, (BT, 1), 0)
    out_ref[...] = jnp.where(rows < E, e, 0.0)


def _combine_body(p_ref, out_ref):
    out_ref[...] = (p_ref[0] + p_ref[1]) * (1.0 / math.sqrt(AVG))


def _full(shape):
    return pl.BlockSpec(shape, lambda i: (0, 0))


def _edge_mlp(rows_s, rows_d, w0b, pt, b0, w1, b1):
    g = EPAD // BT
    return pl.pallas_call(
        _edge_mlp_body,
        grid=(g,),
        in_specs=[
            pl.BlockSpec((BT, 8), lambda i: (i, 0)),
            pl.BlockSpec((BT, 8), lambda i: (i, 0)),
            _full((DS, NB)),
            _full((DS, NT * NT)),
            _full((DS, 1)),
            _full((DS, DS)),
            _full((DS, 1)),
        ],
        out_specs=pl.BlockSpec((BT, DS), lambda i: (i, 0)),
        out_shape=jax.ShapeDtypeStruct((EPAD, DS), jnp.float32),
    )(rows_s, rows_d, w0b, pt, b0, w1, b1)


def _layer(ef, env, a1, a2, lb, lc):
    g = EPAD // BT
    return pl.pallas_call(
        _layer_body,
        grid=(g,),
        in_specs=[
            pl.BlockSpec((BT, DS), lambda i: (i, 0)),
            pl.BlockSpec((BT, DS), lambda i: (i, 0)),
            _full((DS, DS)),
            _full((DS, DS)),
            _full((DS, DS)),
            _full((DS, DS)),
        ],
        out_specs=pl.BlockSpec((BT, DS), lambda i: (i, 0)),
        out_shape=jax.ShapeDtypeStruct((EPAD, DS), jnp.float32),
    )(ef, env, a1, a2, lb, lc)


def _layer2_readout(ef, env, a1, a2, lb, lc, wr0, wr1):
    g = EPAD // BT
    return pl.pallas_call(
        _layer2_readout_body,
        grid=(g,),
        in_specs=[
            pl.BlockSpec((BT, DS), lambda i: (i, 0)),
            pl.BlockSpec((BT, DS), lambda i: (i, 0)),
            _full((DS, DS)),
            _full((DS, DS)),
            _full((DS, DS)),
            _full((DS, DS)),
            _full((8, DS)),
            _full((1, 8)),
        ],
        out_specs=pl.BlockSpec((BT, 1), lambda i: (i, 0)),
        out_shape=jax.ShapeDtypeStruct((EPAD, 1), jnp.float32),
    )(ef, env, a1, a2, lb, lc, wr0, wr1)


def _readout(ef, wr0, wr1):
    g = EPAD // BT
    return pl.pallas_call(
        _readout_body,
        grid=(g,),
        in_specs=[
            pl.BlockSpec((BT, DS), lambda i: (i, 0)),
            _full((8, DS)),
            _full((1, 8)),
        ],
        out_specs=pl.BlockSpec((BT, 1), lambda i: (i, 0)),
        out_shape=jax.ShapeDtypeStruct((EPAD, 1), jnp.float32),
    )(ef, wr0, wr1)


def _combine(partials):
    p3 = partials.reshape(2, NE_PAD // 128, 128)
    return pl.pallas_call(
        _combine_body,
        grid=(1,),
        in_specs=[pl.BlockSpec((2, NE_PAD // 128, 128), lambda i: (0, 0, 0))],
        out_specs=pl.BlockSpec((NE_PAD // 128, 128), lambda i: (0, 0)),
        out_shape=jax.ShapeDtypeStruct((NE_PAD // 128, 128), jnp.float32),
    )(p3)


def kernel(positions, atom_types, edge_index, W_pair, W0, b0, W1, b1,
           L0a, L0b, L0c, L1a, L1b, L1c, Wr0, Wr1):
    src = edge_index[0]
    dst = edge_index[1]
    pad = EPAD - E
    srcp = jnp.concatenate([src, jnp.zeros((pad,), src.dtype)]).astype(jnp.int32)
    dstp = jnp.concatenate([dst, jnp.zeros((pad,), dst.dtype)]).astype(jnp.int32)
    src2 = srcp.reshape(ROWS, 128)
    dst2 = dstp.reshape(ROWS, 128)

    table = jnp.concatenate([positions, atom_types.astype(jnp.float32)[:, None],
                             jnp.zeros((N, 4), jnp.float32)], axis=1)

    both_idx = jnp.concatenate([src2, dst2], axis=0)
    rows_sd = _sc_gather(table, both_idx, 8, 16)
    rows_s = rows_sd[:ROWS].reshape(EPAD, 8)
    rows_d = rows_sd[ROWS:].reshape(EPAD, 8)

    w0bt = W0[:NB].T
    ptt = (W_pair @ W0[NB:]).T
    b0t = b0[:, None]
    b1t = b1[:, None]

    ef = _edge_mlp(rows_s, rows_d, w0bt, ptt, b0t, W1.T, b1t)

    zrows = jnp.zeros((SPAD_ROWS // _NS, DS), jnp.float32)

    node_env = _sc_scatter_env(ef.reshape(ROWS, 128, DS), dst2, zrows)
    env = _sc_gather(node_env, src2, DS, 16).reshape(EPAD, DS)
    ef = _layer(ef, env, L0a[:DS].T, L0a[DS:].T * (1.0 / AVG), L0b.T, L0c.T)

    node_env = _sc_scatter_env(ef.reshape(ROWS, 128, DS), dst2, zrows)
    env = _sc_gather(node_env, src2, DS, 16).reshape(EPAD, DS)
    e = _layer2_readout(ef, env, L1a[:DS].T, L1a[DS:].T * (1.0 / AVG), L1b.T, L1c.T,
                        Wr0.T, Wr1.T)
    z1 = jnp.zeros((NE_PAD // _NS,), jnp.float32)
    partials = _sc_scatter_energy(e.reshape(ROWS, 128), dst2, z1)
    atom_energy = _combine(partials).reshape(NE_PAD)[:N]
    return atom_energy


# speedup vs baseline: 7.1695x; 1.0058x over previous
import functools
import math

import jax
import jax.numpy as jnp
from jax import lax
from jax.experimental import pallas as pl
from jax.experimental.pallas import tpu as pltpu
from jax.experimental.pallas import tpu_sc as plsc

N = 100000
E = 1600000
R_MAX = 4.0
NB = 8
NT = 4
DS = 32
AVG = 20.0

EPAD = 1638400
ROWS = EPAD // 128
BT = 4096

_NC = 2
_NS = 16
_NW = _NC * _NS

HALF = N // 2
SPAD_ROWS = 50048
NE_PAD = 100352


def _silu(x):
    return x / (1.0 + jnp.exp(-x))


def _sc_gather(table, idx2, d, chr_):
    rows = idx2.shape[0]
    rows_per_w = rows // _NW
    n_macro = rows_per_w // chr_
    mesh = plsc.VectorSubcoreMesh(core_axis_name="c", subcore_axis_name="s")

    def body(table_ref, idx_ref, out_ref, idx_v, rows_v, semg, semo):
        wid = lax.axis_index("s") * _NC + lax.axis_index("c")
        base = wid * rows_per_w

        def macro(m, carry):
            b = lax.rem(m, 2)
            rb = base + m * chr_
            pltpu.sync_copy(idx_ref.at[pl.ds(rb, chr_)], idx_v)

            @pl.when(m >= 2)
            def _():
                pltpu.make_async_copy(rows_v.at[b], out_ref.at[pl.ds(rb, chr_)], semo).wait()

            handles = []
            for j in range(chr_):
                handles.append(pltpu.async_copy(table_ref.at[idx_v.at[j]], rows_v.at[b, j], semg))
            for h in handles:
                h.wait()
            pltpu.async_copy(rows_v.at[b], out_ref.at[pl.ds(rb, chr_)], semo)
            return carry

        lax.fori_loop(0, n_macro, macro, 0)
        for b in range(2):
            pltpu.make_async_copy(rows_v.at[b], out_ref.at[pl.ds(base, chr_)], semo).wait()

    f = pl.kernel(
        body,
        out_type=jax.ShapeDtypeStruct((rows, 128, d), jnp.float32),
        mesh=mesh,
        compiler_params=pltpu.CompilerParams(use_tc_tiling_on_sc=False),
        scratch_types=[
            pltpu.VMEM((chr_, 128), jnp.int32),
            pltpu.VMEM((2, chr_, 128, d), jnp.float32),
            pltpu.SemaphoreType.DMA,
            pltpu.SemaphoreType.DMA,
        ],
    )
    return f(table, idx2)


def _sc_scatter_env(ef3, dst2, zrows):
    chr_ = 4
    rows_per_tile = ROWS // _NS
    n_macro = rows_per_tile // chr_
    z_per_tile = SPAD_ROWS // _NS
    o_per_tile = HALF // _NS
    mesh = plsc.VectorSubcoreMesh(core_axis_name="c", subcore_axis_name="s")

    def body(ef_ref, dst_ref, z_ref, out_ref, idx_v, ef_v, acc, sem):
        c = lax.axis_index("c")
        s = lax.axis_index("s")
        pltpu.sync_copy(z_ref.at[pl.ds(0, z_per_tile)], acc.at[pl.ds(s * z_per_tile, z_per_tile)])
        plsc.subcore_barrier()
        lo = c * HALF

        def macro(m, carry):
            rb = s * rows_per_tile + m * chr_
            pltpu.sync_copy(dst_ref.at[pl.ds(rb, chr_)], idx_v)
            pltpu.sync_copy(ef_ref.at[pl.ds(rb, chr_)], ef_v)
            for j in range(chr_):
                for k in range(8):
                    lid = idx_v[j, pl.ds(k * 16, 16)] - lo
                    ok = (lid >= 0) & (lid < HALF)
                    dummy = HALF + (k % 3) * 16 + lax.iota(jnp.int32, 16)
                    idx_v[j, pl.ds(k * 16, 16)] = jnp.where(ok, lid, dummy)
            handles = []
            for j in range(chr_):
                handles.append(pltpu.async_copy(ef_v.at[j], acc.at[idx_v.at[j]], sem, add=True))
            for h in handles:
                h.wait()
            return carry

        lax.fori_loop(0, n_macro, macro, 0)
        plsc.subcore_barrier()
        pltpu.sync_copy(acc.at[pl.ds(s * o_per_tile, o_per_tile)],
                        out_ref.at[pl.ds(c * HALF + s * o_per_tile, o_per_tile)])

    f = pl.kernel(
        body,
        out_type=jax.ShapeDtypeStruct((N, DS), jnp.float32),
        mesh=mesh,
        compiler_params=pltpu.CompilerParams(use_tc_tiling_on_sc=False),
        scratch_types=[
            pltpu.VMEM((chr_, 128), jnp.int32),
            pltpu.VMEM((chr_, 128, DS), jnp.float32),
            pltpu.VMEM_SHARED((SPAD_ROWS, DS), jnp.float32),
            pltpu.SemaphoreType.DMA,
        ],
    )
    return f(ef3, dst2, zrows)


def _sc_scatter_energy(e2, dst2, z1):
    chr_ = 16
    rows_per_tile = ROWS // _NW
    n_macro = rows_per_tile // chr_
    z_per_tile = NE_PAD // _NS
    mesh = plsc.VectorSubcoreMesh(core_axis_name="c", subcore_axis_name="s")

    def body(e_ref, dst_ref, z_ref, out_ref, idx_v, e_v, acc, sem):
        c = lax.axis_index("c")
        s = lax.axis_index("s")
        pltpu.sync_copy(z_ref, acc.at[pl.ds(s * z_per_tile, z_per_tile)])
        plsc.subcore_barrier()
        base = (c * _NS + s) * rows_per_tile

        def macro(m, carry):
            rb = base + m * chr_
            pltpu.sync_copy(dst_ref.at[pl.ds(rb, chr_)], idx_v)
            pltpu.sync_copy(e_ref.at[pl.ds(rb, chr_)], e_v)
            handles = []
            for j in range(chr_):
                handles.append(pltpu.async_copy(e_v.at[j], acc.at[idx_v.at[j]], sem, add=True))
            for h in handles:
                h.wait()
            return carry

        lax.fori_loop(0, n_macro, macro, 0)
        plsc.subcore_barrier()
        pltpu.sync_copy(acc.at[pl.ds(s * z_per_tile, z_per_tile)],
                        out_ref.at[c, pl.ds(s * z_per_tile, z_per_tile)])

    f = pl.kernel(
        body,
        out_type=jax.ShapeDtypeStruct((2, NE_PAD), jnp.float32),
        mesh=mesh,
        compiler_params=pltpu.CompilerParams(use_tc_tiling_on_sc=False),
        scratch_types=[
            pltpu.VMEM((chr_, 128), jnp.int32),
            pltpu.VMEM((chr_, 128), jnp.float32),
            pltpu.VMEM_SHARED((NE_PAD,), jnp.float32),
            pltpu.SemaphoreType.DMA,
        ],
    )
    return f(e2, dst2, z1)


def _edge_mlp_body(rs_ref, rd_ref, w0bt_ref, ptt_ref, b0t_ref, w1t_ref, b1t_ref, out_ref):
    i = pl.program_id(0)
    rst = rs_ref[...].T
    rdt = rd_ref[...].T
    d = rdt - rst
    d2 = d * d
    r2 = d2[0:1] + d2[1:2] + d2[2:3]
    pidf = rst[3:4] * float(NT) + rdt[3:4]
    r = jnp.sqrt(r2 + 1e-12)
    x = r * (1.0 / R_MAX)
    xc = jnp.minimum(x, 1.0)
    t = xc - 0.5
    u2 = (math.pi * math.pi) * (t * t)
    s1m = 1.0 + u2 * (-1.0 / 2 + u2 * (1.0 / 24 + u2 * (-1.0 / 720 + u2 * (1.0 / 40320 + u2 * (-1.0 / 3628800)))))
    c1m = -(math.pi * t) * (1.0 + u2 * (-1.0 / 6 + u2 * (1.0 / 120 + u2 * (-1.0 / 5040 + u2 * (1.0 / 362880 + u2 * (-1.0 / 39916800))))))
    v2 = (math.pi * math.pi) * (xc * xc)
    s1d = (math.pi * xc) * (1.0 + v2 * (-1.0 / 6 + v2 * (1.0 / 120 + v2 * (-1.0 / 5040))))
    c1d = 1.0 + v2 * (-1.0 / 2 + v2 * (1.0 / 24 + v2 * (-1.0 / 720 + v2 * (1.0 / 40320))))
    small = xc < 0.25
    s1 = jnp.where(small, s1d, s1m)
    c1 = jnp.where(small, c1d, c1m)
    tc1 = c1 + c1
    s2 = tc1 * s1
    s3 = tc1 * s2 - s1
    s4 = tc1 * s3 - s2
    s5 = tc1 * s4 - s3
    s6 = tc1 * s5 - s4
    s7 = tc1 * s6 - s5
    s8 = tc1 * s7 - s6
    bes = jnp.concatenate([s1, s2, s3, s4, s5, s6, s7, s8], axis=0)
    p = 6.0
    x2 = x * x
    x6 = x2 * x2 * x2
    cut = 1.0 - (p + 1.0) * (p + 2.0) / 2.0 * x6 + p * (p + 2.0) * x6 * x - p * (p + 1.0) / 2.0 * x6 * x2
    cut = jnp.where(x < 1.0, cut, 0.0)
    pref = (math.sqrt(2.0 / R_MAX) * cut) / (r + 1e-9)
    bes = bes * pref
    onehot = (jax.lax.broadcasted_iota(jnp.int32, (NT * NT, BT), 0).astype(jnp.float32) == pidf
              ).astype(jnp.float32)
    h = jnp.dot(w0bt_ref[...], bes, preferred_element_type=jnp.float32)
    h = h + jnp.dot(ptt_ref[...], onehot, preferred_element_type=jnp.float32)
    h = _silu(h + b0t_ref[...])
    eft = _silu(jnp.dot(w1t_ref[...], h, preferred_element_type=jnp.float32) + b1t_ref[...])
    cols = jax.lax.broadcasted_iota(jnp.int32, (1, BT), 1)
    eft = jnp.where(cols < (E - i * BT), eft, 0.0)
    out_ref[...] = eft.T


def _layer_body(ef_ref, env_ref, a1_ref, a2_ref, lb_ref, lc_ref, out_ref):
    i = pl.program_id(0)
    eft = ef_ref[...].T
    envt = env_ref[...].T
    z = _silu(jnp.dot(a1_ref[...], eft, preferred_element_type=jnp.float32)
              + jnp.dot(a2_ref[...], envt, preferred_element_type=jnp.float32))
    z = _silu(jnp.dot(lb_ref[...], z, preferred_element_type=jnp.float32))
    z = _silu(jnp.dot(lc_ref[...], z, preferred_element_type=jnp.float32))
    out = eft + z
    cols = jax.lax.broadcasted_iota(jnp.int32, (1, BT), 1)
    out = jnp.where(cols < (E - i * BT), out, 0.0)
    out_ref[...] = out.T


def _layer2_readout_body(ef_ref, env_ref, a1_ref, a2_ref, lb_ref, lc_ref,
                         wr0_ref, wr1_ref, out_ref):
    i = pl.program_id(0)
    eft = ef_ref[...].T
    envt = env_ref[...].T
    z = _silu(jnp.dot(a1_ref[...], eft, preferred_element_type=jnp.float32)
              + jnp.dot(a2_ref[...], envt, preferred_element_type=jnp.float32))
    z = _silu(jnp.dot(lb_ref[...], z, preferred_element_type=jnp.float32))
    z = _silu(jnp.dot(lc_ref[...], z, preferred_element_type=jnp.float32))
    eft = eft + z
    s = _silu(jnp.dot(wr0_ref[...], eft, preferred_element_type=jnp.float32))
    e = jnp.dot(wr1_ref[...], s, preferred_element_type=jnp.float32)
    cols = jax.lax.broadcasted_iota(jnp.int32, (1, BT), 1)
    e = jnp.where(cols < (E - i * BT), e, 0.0)
    out_ref[...] = e.T


def _readout_body(ef_ref, wr0_ref, wr1_ref, out_ref):
    i = pl.program_id(0)
    s = _silu(jnp.dot(ef_ref[...], wr0_ref[...], preferred_element_type=jnp.float32))
    e = jnp.dot(s, wr1_ref[...], preferred_element_type=jnp.float32)
    rows = i * BT + jax.lax.broadcasted_iota(jnp.int32, (BT, 1), 0)
    out_ref[...] = jnp.where(rows < E, e, 0.0)


def _combine_body(p_ref, out_ref):
    out_ref[...] = (p_ref[0] + p_ref[1]) * (1.0 / math.sqrt(AVG))


def _full(shape):
    return pl.BlockSpec(shape, lambda i: (0, 0))


def _edge_mlp(rows_s, rows_d, w0b, pt, b0, w1, b1):
    g = EPAD // BT
    return pl.pallas_call(
        _edge_mlp_body,
        grid=(g,),
        in_specs=[
            pl.BlockSpec((BT, 8), lambda i: (i, 0)),
            pl.BlockSpec((BT, 8), lambda i: (i, 0)),
            _full((DS, NB)),
            _full((DS, NT * NT)),
            _full((DS, 1)),
            _full((DS, DS)),
            _full((DS, 1)),
        ],
        out_specs=pl.BlockSpec((BT, DS), lambda i: (i, 0)),
        out_shape=jax.ShapeDtypeStruct((EPAD, DS), jnp.float32),
    )(rows_s, rows_d, w0b, pt, b0, w1, b1)


def _layer(ef, env, a1, a2, lb, lc):
    g = EPAD // BT
    return pl.pallas_call(
        _layer_body,
        grid=(g,),
        in_specs=[
            pl.BlockSpec((BT, DS), lambda i: (i, 0)),
            pl.BlockSpec((BT, DS), lambda i: (i, 0)),
            _full((DS, DS)),
            _full((DS, DS)),
            _full((DS, DS)),
            _full((DS, DS)),
        ],
        out_specs=pl.BlockSpec((BT, DS), lambda i: (i, 0)),
        out_shape=jax.ShapeDtypeStruct((EPAD, DS), jnp.float32),
    )(ef, env, a1, a2, lb, lc)


def _layer2_readout(ef, env, a1, a2, lb, lc, wr0, wr1):
    g = EPAD // BT
    return pl.pallas_call(
        _layer2_readout_body,
        grid=(g,),
        in_specs=[
            pl.BlockSpec((BT, DS), lambda i: (i, 0)),
            pl.BlockSpec((BT, DS), lambda i: (i, 0)),
            _full((DS, DS)),
            _full((DS, DS)),
            _full((DS, DS)),
            _full((DS, DS)),
            _full((8, DS)),
            _full((1, 8)),
        ],
        out_specs=pl.BlockSpec((BT, 1), lambda i: (i, 0)),
        out_shape=jax.ShapeDtypeStruct((EPAD, 1), jnp.float32),
    )(ef, env, a1, a2, lb, lc, wr0, wr1)


def _readout(ef, wr0, wr1):
    g = EPAD // BT
    return pl.pallas_call(
        _readout_body,
        grid=(g,),
        in_specs=[
            pl.BlockSpec((BT, DS), lambda i: (i, 0)),
            _full((8, DS)),
            _full((1, 8)),
        ],
        out_specs=pl.BlockSpec((BT, 1), lambda i: (i, 0)),
        out_shape=jax.ShapeDtypeStruct((EPAD, 1), jnp.float32),
    )(ef, wr0, wr1)


def _combine(partials):
    p3 = partials.reshape(2, NE_PAD // 128, 128)
    return pl.pallas_call(
        _combine_body,
        grid=(1,),
        in_specs=[pl.BlockSpec((2, NE_PAD // 128, 128), lambda i: (0, 0, 0))],
        out_specs=pl.BlockSpec((NE_PAD // 128, 128), lambda i: (0, 0)),
        out_shape=jax.ShapeDtypeStruct((NE_PAD // 128, 128), jnp.float32),
    )(p3)


def kernel(positions, atom_types, edge_index, W_pair, W0, b0, W1, b1,
           L0a, L0b, L0c, L1a, L1b, L1c, Wr0, Wr1):
    src = edge_index[0]
    dst = edge_index[1]
    pad = EPAD - E
    srcp = jnp.concatenate([src, jnp.zeros((pad,), src.dtype)]).astype(jnp.int32)
    dstp = jnp.concatenate([dst, jnp.zeros((pad,), dst.dtype)]).astype(jnp.int32)
    src2 = srcp.reshape(ROWS, 128)
    dst2 = dstp.reshape(ROWS, 128)

    table = jnp.concatenate([positions, atom_types.astype(jnp.float32)[:, None],
                             jnp.zeros((N, 4), jnp.float32)], axis=1)

    both_idx = jnp.concatenate([src2, dst2], axis=0)
    rows_sd = _sc_gather(table, both_idx, 8, 16)
    rows_s = rows_sd[:ROWS].reshape(EPAD, 8)
    rows_d = rows_sd[ROWS:].reshape(EPAD, 8)

    w0bt = W0[:NB].T
    ptt = (W_pair @ W0[NB:]).T
    b0t = b0[:, None]
    b1t = b1[:, None]

    ef = _edge_mlp(rows_s, rows_d, w0bt, ptt, b0t, W1.T, b1t)

    zrows = jnp.zeros((SPAD_ROWS // _NS, DS), jnp.float32)

    node_env = _sc_scatter_env(ef.reshape(ROWS, 128, DS), dst2, zrows)
    env = _sc_gather(node_env, src2, DS, 8).reshape(EPAD, DS)
    ef = _layer(ef, env, L0a[:DS].T, L0a[DS:].T * (1.0 / AVG), L0b.T, L0c.T)

    node_env = _sc_scatter_env(ef.reshape(ROWS, 128, DS), dst2, zrows)
    env = _sc_gather(node_env, src2, DS, 8).reshape(EPAD, DS)
    e = _layer2_readout(ef, env, L1a[:DS].T, L1a[DS:].T * (1.0 / AVG), L1b.T, L1c.T,
                        Wr0.T, Wr1.T)
    z1 = jnp.zeros((NE_PAD // _NS,), jnp.float32)
    partials = _sc_scatter_energy(e.reshape(ROWS, 128), dst2, z1)
    atom_energy = _combine(partials).reshape(NE_PAD)[:N]
    return atom_energy
